# baseline (speedup 1.0000x reference)
import functools

import jax
import jax.numpy as jnp
from jax import lax
from jax.experimental import pallas as pl
from jax.experimental.pallas import tpu as pltpu
from jax.experimental.pallas import tpu_sc as plsc


def _gather_sc(idx, table, cb):
    n, = idx.shape
    v, d = table.shape
    assert d == 32
    info = plsc.get_sparse_core_info()
    nc = info.num_cores
    nw = nc * info.num_subcores
    n_per_w = n // nw
    blocks = n // 128
    blocks_per_w = blocks // nw
    a = cb * 128
    n_chunks = n_per_w // a
    mesh = plsc.VectorSubcoreMesh(core_axis_name="c", subcore_axis_name="s")

    @functools.partial(
        pl.kernel,
        mesh=mesh,
        out_type=jax.ShapeDtypeStruct((d // 8, blocks, 8, 128), jnp.float32),
        scratch_types=[
            pltpu.VMEM((a,), jnp.int32),
            pltpu.VMEM((a,), jnp.int32),
            pltpu.VMEM((a, d), jnp.float32),
            pltpu.VMEM((a, d), jnp.float32),
            pltpu.VMEM((d // 8, cb, 8, 128), jnp.float32),
            pltpu.SemaphoreType.DMA,
            pltpu.SemaphoreType.DMA,
            pltpu.SemaphoreType.DMA,
            pltpu.SemaphoreType.DMA,
            pltpu.SemaphoreType.DMA,
            pltpu.SemaphoreType.DMA,
        ],
        compiler_params=pltpu.CompilerParams(
            use_tc_tiling_on_sc=False, needs_layout_passes=False),
    )
    def k(idx_hbm, table_hbm, out_hbm, idx_v0, idx_v1, rows_v0, rows_v1,
          rows_t, si0, si1, sg0, sg1, so0, so1):
        wid = lax.axis_index("s") * nc + lax.axis_index("c")
        base = wid * n_per_w
        base_blk = wid * blocks_per_w
        idx_v = (idx_v0, idx_v1)
        rows_v = (rows_v0, rows_v1)
        si = (si0, si1)
        sg = (sg0, sg1)
        so = (so0, so1)

        jj = jnp.arange(16, dtype=jnp.int32)
        g_lo = jj >> 3
        g_hi = g_lo + 2
        s_pat = jj & 7

        def transpose_chunk(src):
            @pl.loop(0, a, unroll=8)
            def _(au):
                c = au >> 7
                l = au & 127
                cv = jnp.full((16,), c, jnp.int32)
                lv = jnp.full((16,), l, jnp.int32)
                v0 = src[au, pl.ds(0, 16)]
                v1 = src[au, pl.ds(16, 16)]
                plsc.store_scatter(rows_t, [g_lo, cv, s_pat, lv], v0)
                plsc.store_scatter(rows_t, [g_hi, cv, s_pat, lv], v1)

        idx_d = [None, None]
        gat_d = [None, None]
        out_d = [None, None, None, None]

        for b in range(min(2, n_chunks)):
            idx_d[b] = pltpu.async_copy(
                idx_hbm.at[pl.ds(base + b * a, a)], idx_v[b], si[b])

        def drain_and_emit(j):
            p = j % 2
            for g in range(4):
                if out_d[g] is not None:
                    out_d[g].wait()
            transpose_chunk(rows_v[p])
            for g in range(4):
                out_d[g] = pltpu.async_copy(
                    rows_t.at[g],
                    out_hbm.at[g, pl.ds(base_blk + j * cb, cb)],
                    so[g % 2])

        for j in range(n_chunks):
            b = j % 2
            idx_d[b].wait()
            gat_d[b] = pltpu.async_copy(
                table_hbm.at[idx_v[b]], rows_v[b], sg[b])
            if j >= 1:
                p = (j - 1) % 2
                gat_d[p].wait()
                if j + 1 < n_chunks:
                    idx_d[p] = pltpu.async_copy(
                        idx_hbm.at[pl.ds(base + (j + 1) * a, a)],
                        idx_v[p], si[p])
                drain_and_emit(j - 1)

        gat_d[(n_chunks - 1) % 2].wait()
        drain_and_emit(n_chunks - 1)
        for g in range(4):
            out_d[g].wait()

    return k(idx, table)


def kernel(inputs, table):
    bsz, h = inputs.shape
    _, d = table.shape
    idx = inputs.reshape(bsz * h).astype(jnp.int32)
    out4d = _gather_sc(idx, table, cb=10)
    return out4d.transpose(1, 3, 0, 2).reshape(bsz * h, d)
